# baseline (speedup 1.0000x reference)
import jax
import jax.numpy as jnp
from jax import lax
from jax.experimental import pallas as pl
from jax.experimental.pallas import tpu as pltpu
from jax.experimental.pallas import tpu_sc as plsc

N = 10000
D = 128
NP = 10240
ES = 320000 + NP
CH = 64
EPT = -(-ES // (16 * 128)) * 128
E_PAD = 16 * EPT
NCH = EPT // CH
SB = 6
B = 1024
GRID = NP // B

_mesh = plsc.VectorSubcoreMesh(core_axis_name="c", subcore_axis_name="s")


def _edge_body(h_hbm, als_hbm, ald_hbm, src_hbm, dst_hbm, u_hbm, den_hbm,
               su, als_v, ald_v, den_v, src_c, dst_c, gidx_v, didx_v0,
               didx_v1, w_v, rows_v0, rows_v1, sem, sem2):
    c = lax.axis_index("c")
    s = lax.axis_index("s")
    wid = c * 16 + s
    coff = c * NP

    pltpu.sync_copy(als_hbm.at[pl.ds(coff, NP)], als_v)
    pltpu.sync_copy(ald_hbm.at[pl.ds(coff, NP)], ald_v)

    zero = jnp.zeros((16,), jnp.float32)

    @pl.loop(0, CH)
    def _(i):
        for j in range(8):
            rows_v0[i, pl.ds(j * 16, 16)] = zero

    @pl.loop(0, NP // 16)
    def _(i):
        den_v[pl.ds(i * 16, 16)] = zero

    @pl.loop(0, NP // 16 // CH)
    def _(i):
        r0 = s * (NP // 16) + i * CH
        pltpu.sync_copy(rows_v0, su.at[pl.ds(r0, CH)])

    plsc.subcore_barrier()

    rows_bufs = (rows_v0, rows_v1)
    didx_bufs = (didx_v0, didx_v1)

    @pl.loop(0, NCH // SB)
    def _(sb):
        sbase = sb * SB * CH
        pltpu.sync_copy(src_hbm.at[wid, pl.ds(sbase, SB * CH)], src_c)
        pltpu.sync_copy(dst_hbm.at[wid, pl.ds(sbase, SB * CH)], dst_c)

        @pl.loop(0, SB // 2)
        def _(p):
            for b in range(2):
                ci = p * 2 + b
                base = ci * CH
                gci = sb * SB + ci
                rows_b = rows_bufs[b]
                didx_b = didx_bufs[b]

                @pl.loop(0, CH // 16)
                def _(g):
                    off = base + g * 16
                    gidx_v[pl.ds(g * 16, 16)] = src_c[pl.ds(off, 16)] + coff

                @pl.when(gci >= 2)
                def _():
                    pltpu.make_async_copy(rows_b, su.at[didx_b], sem2).wait()

                cp = pltpu.async_copy(h_hbm.at[gidx_v], rows_b, sem)

                @pl.loop(0, CH // 16)
                def _(g):
                    off = base + g * 16
                    si = src_c[pl.ds(off, 16)]
                    di = dst_c[pl.ds(off, 16)]
                    didx_b[pl.ds(g * 16, 16)] = di
                    a = (plsc.load_gather(als_v, [si])
                         + plsc.load_gather(ald_v, [di]))
                    a = jnp.where(a > 0.0, a, 0.2 * a)
                    w = jnp.exp(a)
                    w_v[pl.ds(g * 16, 16)] = w
                    plsc.addupdate_scatter(den_v, [di], w)

                cp.wait()

                @pl.loop(0, CH, unroll=4)
                def _(e):
                    wv = plsc.load_gather(w_v, [jnp.full((16,), e, jnp.int32)])
                    for j in range(8):
                        rows_b[e, pl.ds(j * 16, 16)] = (
                            rows_b[e, pl.ds(j * 16, 16)] * wv)

                pltpu.async_copy(rows_b, su.at[didx_b], sem2, add=True)

    pltpu.make_async_copy(rows_v0, su.at[didx_v0], sem2).wait()
    pltpu.make_async_copy(rows_v1, su.at[didx_v1], sem2).wait()

    plsc.subcore_barrier()

    @pl.loop(0, NP // 16 // CH)
    def _(i):
        r0 = s * (NP // 16) + i * CH
        pltpu.sync_copy(su.at[pl.ds(r0, CH)], rows_v0)
        pltpu.sync_copy(rows_v0, u_hbm.at[pl.ds(coff + r0, CH)])
    pltpu.sync_copy(den_v, den_hbm.at[wid])


_edge_kernel = pl.kernel(
    _edge_body,
    out_type=[
        jax.ShapeDtypeStruct((2 * NP, D), jnp.float32),
        jax.ShapeDtypeStruct((32, NP), jnp.float32),
    ],
    mesh=_mesh,
    scratch_types=[
        pltpu.VMEM_SHARED((NP, D), jnp.float32),
        pltpu.VMEM((NP,), jnp.float32),
        pltpu.VMEM((NP,), jnp.float32),
        pltpu.VMEM((NP,), jnp.float32),
        pltpu.VMEM((SB * CH,), jnp.int32),
        pltpu.VMEM((SB * CH,), jnp.int32),
        pltpu.VMEM((CH,), jnp.int32),
        pltpu.VMEM((CH,), jnp.int32),
        pltpu.VMEM((CH,), jnp.int32),
        pltpu.VMEM((CH,), jnp.float32),
        pltpu.VMEM((CH, D), jnp.float32),
        pltpu.VMEM((CH, D), jnp.float32),
        pltpu.SemaphoreType.DMA,
        pltpu.SemaphoreType.DMA,
    ],
    compiler_params=pltpu.CompilerParams(needs_layout_passes=False),
)


def _t1_body(x_ref, w_ref, am_ref, h_ref, al_ref):
    h = jnp.dot(x_ref[...], w_ref[...], preferred_element_type=jnp.float32)
    h_ref[0] = h[:, :D]
    h_ref[1] = h[:, D:]
    al_ref[...] = jnp.dot(h, am_ref[...], preferred_element_type=jnp.float32)


def _dcol(den_ref, k):
    return jnp.sum(den_ref[k], axis=0)[:, None] + 1e-16


def _t2_body(u_ref, den_ref, b_ref, w2_ref, am_ref, h_ref, al_ref):
    x1o = jax.nn.relu(u_ref[0] / _dcol(den_ref, 0) + b_ref[0])
    x1s = jax.nn.relu(u_ref[1] / _dcol(den_ref, 1) + b_ref[1])
    x1 = jnp.concatenate([x1o, x1s], axis=1)
    h = jnp.dot(x1, w2_ref[...], preferred_element_type=jnp.float32)
    h_ref[0] = h[:, :D]
    h_ref[1] = h[:, D:]
    al_ref[...] = jnp.dot(h, am_ref[...], preferred_element_type=jnp.float32)


def _t3_body(u_ref, den_ref, b_ref, wp_ref, bp_ref, out_ref):
    x2o = u_ref[0] / _dcol(den_ref, 0) + b_ref[0]
    x2s = u_ref[1] / _dcol(den_ref, 1) + b_ref[1]
    x2 = jnp.concatenate([x2o, x2s], axis=1)
    out_ref[...] = (jnp.dot(x2, wp_ref[...], preferred_element_type=jnp.float32)
                    + bp_ref[...])


_u_spec = pl.BlockSpec((2, B, D), lambda i: (0, i, 0))
_den_spec = pl.BlockSpec((2, 16, B), lambda i: (0, 0, i))
_b_spec = pl.BlockSpec((2, 1, D), lambda i: (0, 0, 0))

_t1_call = pl.pallas_call(
    _t1_body,
    grid=(GRID,),
    in_specs=[
        pl.BlockSpec((B, D), lambda i: (i, 0)),
        pl.BlockSpec((D, 2 * D), lambda i: (0, 0)),
        pl.BlockSpec((2 * D, 8), lambda i: (0, 0)),
    ],
    out_specs=[
        pl.BlockSpec((2, B, D), lambda i: (0, i, 0)),
        pl.BlockSpec((B, 8), lambda i: (i, 0)),
    ],
    out_shape=[
        jax.ShapeDtypeStruct((2, NP, D), jnp.float32),
        jax.ShapeDtypeStruct((NP, 8), jnp.float32),
    ],
)

_t2_call = pl.pallas_call(
    _t2_body,
    grid=(GRID,),
    in_specs=[
        _u_spec, _den_spec, _b_spec,
        pl.BlockSpec((2 * D, 2 * D), lambda i: (0, 0)),
        pl.BlockSpec((2 * D, 8), lambda i: (0, 0)),
    ],
    out_specs=[
        pl.BlockSpec((2, B, D), lambda i: (0, i, 0)),
        pl.BlockSpec((B, 8), lambda i: (i, 0)),
    ],
    out_shape=[
        jax.ShapeDtypeStruct((2, NP, D), jnp.float32),
        jax.ShapeDtypeStruct((NP, 8), jnp.float32),
    ],
)

_t3_call = pl.pallas_call(
    _t3_body,
    grid=(GRID,),
    in_specs=[
        _u_spec, _den_spec, _b_spec,
        pl.BlockSpec((2 * D, D), lambda i: (0, 0)),
        pl.BlockSpec((1, D), lambda i: (0, 0)),
    ],
    out_specs=pl.BlockSpec((B, D), lambda i: (i, 0)),
    out_shape=jax.ShapeDtypeStruct((NP, D), jnp.float32),
)


def _amat(a_list):
    z = jnp.zeros((D,), jnp.float32)
    cols = [jnp.concatenate([a_list[0], z]), jnp.concatenate([a_list[1], z]),
            jnp.concatenate([z, a_list[2]]), jnp.concatenate([z, a_list[3]])]
    cols += [jnp.concatenate([z, z])] * 4
    return jnp.stack(cols, axis=1)


def kernel(x, edge_index_o, edge_index_s, W_o1, as_o1, ad_o1, b_o1,
           W_s1, as_s1, ad_s1, b_s1, W_o2, as_o2, ad_o2, b_o2,
           W_s2, as_s2, ad_s2, b_s2, W_pred, b_pred):
    loops = jnp.arange(NP, dtype=jnp.int32)

    def prep(ei):
        s = jnp.concatenate([ei[0], loops])
        d = jnp.concatenate([ei[1], loops])
        s = jnp.pad(s, (0, E_PAD - ES), constant_values=N)
        d = jnp.pad(d, (0, E_PAD - ES), constant_values=N)
        return s.reshape(16, EPT), d.reshape(16, EPT)

    so, do_ = prep(edge_index_o)
    ss, ds_ = prep(edge_index_s)
    src = jnp.concatenate([so, ss], axis=0)
    dst = jnp.concatenate([do_, ds_], axis=0)

    xp = jnp.pad(x, ((0, NP - N), (0, 0)))

    w1 = jnp.concatenate([W_o1, W_s1], axis=1)
    am1 = _amat([as_o1, ad_o1, as_s1, ad_s1])
    h1, al1 = _t1_call(xp, w1, am1)

    als1 = jnp.concatenate([al1[:, 0], al1[:, 2]])
    ald1 = jnp.concatenate([al1[:, 1], al1[:, 3]])

    u1, den1 = _edge_kernel(h1.reshape(2 * NP, D), als1, ald1, src, dst)

    b1 = jnp.stack([b_o1, b_s1]).reshape(2, 1, D)
    w2 = jnp.concatenate([W_o2, W_s2], axis=1)
    am2 = _amat([as_o2, ad_o2, as_s2, ad_s2])
    h2, al2 = _t2_call(u1.reshape(2, NP, D), den1.reshape(2, 16, NP),
                       b1, w2, am2)

    als2 = jnp.concatenate([al2[:, 0], al2[:, 2]])
    ald2 = jnp.concatenate([al2[:, 1], al2[:, 3]])

    u2, den2 = _edge_kernel(h2.reshape(2 * NP, D), als2, ald2, src, dst)

    b2 = jnp.stack([b_o2, b_s2]).reshape(2, 1, D)
    out = _t3_call(u2.reshape(2, NP, D), den2.reshape(2, 16, NP),
                   b2, W_pred, b_pred.reshape(1, D))
    return out[:N]


# speedup vs baseline: 1.0341x; 1.0341x over previous
import jax
import jax.numpy as jnp
from jax import lax
from jax.experimental import pallas as pl
from jax.experimental.pallas import tpu as pltpu
from jax.experimental.pallas import tpu_sc as plsc

N = 10000
D = 128
NP = 10240
ES = 320000 + NP
CH = 64
EPT = -(-ES // (16 * 128)) * 128
E_PAD = 16 * EPT
NCH = EPT // CH
SB = 12
B = 1024
GRID = NP // B

_mesh = plsc.VectorSubcoreMesh(core_axis_name="c", subcore_axis_name="s")


def _edge_body(h_hbm, als_hbm, ald_hbm, src_hbm, dst_hbm, u_hbm, den_hbm,
               su, als_v, ald_v, den_v, src_c, dst_c, gidx_v, didx_v0,
               didx_v1, w_v, rows_v0, rows_v1, sem, sem2):
    c = lax.axis_index("c")
    s = lax.axis_index("s")
    wid = c * 16 + s
    coff = c * NP

    pltpu.sync_copy(als_hbm.at[pl.ds(coff, NP)], als_v)
    pltpu.sync_copy(ald_hbm.at[pl.ds(coff, NP)], ald_v)

    zero = jnp.zeros((16,), jnp.float32)

    @pl.loop(0, CH)
    def _(i):
        for j in range(8):
            rows_v0[i, pl.ds(j * 16, 16)] = zero

    @pl.loop(0, NP // 16)
    def _(i):
        den_v[pl.ds(i * 16, 16)] = zero

    @pl.loop(0, NP // 16 // CH)
    def _(i):
        r0 = s * (NP // 16) + i * CH
        pltpu.sync_copy(rows_v0, su.at[pl.ds(r0, CH)])

    plsc.subcore_barrier()

    rows_bufs = (rows_v0, rows_v1)
    didx_bufs = (didx_v0, didx_v1)

    @pl.loop(0, NCH // SB)
    def _(sb):
        sbase = sb * SB * CH
        pltpu.sync_copy(src_hbm.at[wid, pl.ds(sbase, SB * CH)], src_c)
        pltpu.sync_copy(dst_hbm.at[wid, pl.ds(sbase, SB * CH)], dst_c)

        @pl.loop(0, SB // 2)
        def _(p):
            for b in range(2):
                ci = p * 2 + b
                base = ci * CH
                gci = sb * SB + ci
                rows_b = rows_bufs[b]
                didx_b = didx_bufs[b]

                for g in range(CH // 16):
                    off = base + g * 16
                    gidx_v[pl.ds(g * 16, 16)] = src_c[pl.ds(off, 16)] + coff

                @pl.when(gci >= 2)
                def _():
                    pltpu.make_async_copy(rows_b, su.at[didx_b], sem2).wait()

                cp = pltpu.async_copy(h_hbm.at[gidx_v], rows_b, sem)

                for g in range(CH // 16):
                    off = base + g * 16
                    si = src_c[pl.ds(off, 16)]
                    di = dst_c[pl.ds(off, 16)]
                    didx_b[pl.ds(g * 16, 16)] = di
                    a = (plsc.load_gather(als_v, [si])
                         + plsc.load_gather(ald_v, [di]))
                    a = jnp.where(a > 0.0, a, 0.2 * a)
                    w = jnp.exp(a)
                    w_v[pl.ds(g * 16, 16)] = w
                    plsc.addupdate_scatter(den_v, [di], w)

                cp.wait()

                @pl.loop(0, CH, unroll=4)
                def _(e):
                    wv = plsc.load_gather(w_v, [jnp.full((16,), e, jnp.int32)])
                    for j in range(8):
                        rows_b[e, pl.ds(j * 16, 16)] = (
                            rows_b[e, pl.ds(j * 16, 16)] * wv)

                pltpu.async_copy(rows_b, su.at[didx_b], sem2, add=True)

    pltpu.make_async_copy(rows_v0, su.at[didx_v0], sem2).wait()
    pltpu.make_async_copy(rows_v1, su.at[didx_v1], sem2).wait()

    plsc.subcore_barrier()

    @pl.loop(0, NP // 16 // CH)
    def _(i):
        r0 = s * (NP // 16) + i * CH
        pltpu.sync_copy(su.at[pl.ds(r0, CH)], rows_v0)
        pltpu.sync_copy(rows_v0, u_hbm.at[pl.ds(coff + r0, CH)])
    pltpu.sync_copy(den_v, den_hbm.at[wid])


_edge_kernel = pl.kernel(
    _edge_body,
    out_type=[
        jax.ShapeDtypeStruct((2 * NP, D), jnp.float32),
        jax.ShapeDtypeStruct((32, NP), jnp.float32),
    ],
    mesh=_mesh,
    scratch_types=[
        pltpu.VMEM_SHARED((NP, D), jnp.float32),
        pltpu.VMEM((NP,), jnp.float32),
        pltpu.VMEM((NP,), jnp.float32),
        pltpu.VMEM((NP,), jnp.float32),
        pltpu.VMEM((SB * CH,), jnp.int32),
        pltpu.VMEM((SB * CH,), jnp.int32),
        pltpu.VMEM((CH,), jnp.int32),
        pltpu.VMEM((CH,), jnp.int32),
        pltpu.VMEM((CH,), jnp.int32),
        pltpu.VMEM((CH,), jnp.float32),
        pltpu.VMEM((CH, D), jnp.float32),
        pltpu.VMEM((CH, D), jnp.float32),
        pltpu.SemaphoreType.DMA,
        pltpu.SemaphoreType.DMA,
    ],
    compiler_params=pltpu.CompilerParams(needs_layout_passes=False),
)


def _t1_body(x_ref, w_ref, am_ref, h_ref, al_ref):
    h = jnp.dot(x_ref[...], w_ref[...], preferred_element_type=jnp.float32)
    h_ref[0] = h[:, :D]
    h_ref[1] = h[:, D:]
    al_ref[...] = jnp.dot(h, am_ref[...], preferred_element_type=jnp.float32)


def _dcol(den_ref, k):
    return jnp.sum(den_ref[k], axis=0)[:, None] + 1e-16


def _t2_body(u_ref, den_ref, b_ref, w2_ref, am_ref, h_ref, al_ref):
    x1o = jax.nn.relu(u_ref[0] / _dcol(den_ref, 0) + b_ref[0])
    x1s = jax.nn.relu(u_ref[1] / _dcol(den_ref, 1) + b_ref[1])
    x1 = jnp.concatenate([x1o, x1s], axis=1)
    h = jnp.dot(x1, w2_ref[...], preferred_element_type=jnp.float32)
    h_ref[0] = h[:, :D]
    h_ref[1] = h[:, D:]
    al_ref[...] = jnp.dot(h, am_ref[...], preferred_element_type=jnp.float32)


def _t3_body(u_ref, den_ref, b_ref, wp_ref, bp_ref, out_ref):
    x2o = u_ref[0] / _dcol(den_ref, 0) + b_ref[0]
    x2s = u_ref[1] / _dcol(den_ref, 1) + b_ref[1]
    x2 = jnp.concatenate([x2o, x2s], axis=1)
    out_ref[...] = (jnp.dot(x2, wp_ref[...], preferred_element_type=jnp.float32)
                    + bp_ref[...])


_u_spec = pl.BlockSpec((2, B, D), lambda i: (0, i, 0))
_den_spec = pl.BlockSpec((2, 16, B), lambda i: (0, 0, i))
_b_spec = pl.BlockSpec((2, 1, D), lambda i: (0, 0, 0))

_t1_call = pl.pallas_call(
    _t1_body,
    grid=(GRID,),
    in_specs=[
        pl.BlockSpec((B, D), lambda i: (i, 0)),
        pl.BlockSpec((D, 2 * D), lambda i: (0, 0)),
        pl.BlockSpec((2 * D, 8), lambda i: (0, 0)),
    ],
    out_specs=[
        pl.BlockSpec((2, B, D), lambda i: (0, i, 0)),
        pl.BlockSpec((B, 8), lambda i: (i, 0)),
    ],
    out_shape=[
        jax.ShapeDtypeStruct((2, NP, D), jnp.float32),
        jax.ShapeDtypeStruct((NP, 8), jnp.float32),
    ],
)

_t2_call = pl.pallas_call(
    _t2_body,
    grid=(GRID,),
    in_specs=[
        _u_spec, _den_spec, _b_spec,
        pl.BlockSpec((2 * D, 2 * D), lambda i: (0, 0)),
        pl.BlockSpec((2 * D, 8), lambda i: (0, 0)),
    ],
    out_specs=[
        pl.BlockSpec((2, B, D), lambda i: (0, i, 0)),
        pl.BlockSpec((B, 8), lambda i: (i, 0)),
    ],
    out_shape=[
        jax.ShapeDtypeStruct((2, NP, D), jnp.float32),
        jax.ShapeDtypeStruct((NP, 8), jnp.float32),
    ],
)

_t3_call = pl.pallas_call(
    _t3_body,
    grid=(GRID,),
    in_specs=[
        _u_spec, _den_spec, _b_spec,
        pl.BlockSpec((2 * D, D), lambda i: (0, 0)),
        pl.BlockSpec((1, D), lambda i: (0, 0)),
    ],
    out_specs=pl.BlockSpec((B, D), lambda i: (i, 0)),
    out_shape=jax.ShapeDtypeStruct((NP, D), jnp.float32),
)


def _amat(a_list):
    z = jnp.zeros((D,), jnp.float32)
    cols = [jnp.concatenate([a_list[0], z]), jnp.concatenate([a_list[1], z]),
            jnp.concatenate([z, a_list[2]]), jnp.concatenate([z, a_list[3]])]
    cols += [jnp.concatenate([z, z])] * 4
    return jnp.stack(cols, axis=1)


def kernel(x, edge_index_o, edge_index_s, W_o1, as_o1, ad_o1, b_o1,
           W_s1, as_s1, ad_s1, b_s1, W_o2, as_o2, ad_o2, b_o2,
           W_s2, as_s2, ad_s2, b_s2, W_pred, b_pred):
    loops = jnp.arange(NP, dtype=jnp.int32)

    def prep(ei):
        s = jnp.concatenate([ei[0], loops])
        d = jnp.concatenate([ei[1], loops])
        s = jnp.pad(s, (0, E_PAD - ES), constant_values=N)
        d = jnp.pad(d, (0, E_PAD - ES), constant_values=N)
        return s.reshape(16, EPT), d.reshape(16, EPT)

    so, do_ = prep(edge_index_o)
    ss, ds_ = prep(edge_index_s)
    src = jnp.concatenate([so, ss], axis=0)
    dst = jnp.concatenate([do_, ds_], axis=0)

    xp = jnp.pad(x, ((0, NP - N), (0, 0)))

    w1 = jnp.concatenate([W_o1, W_s1], axis=1)
    am1 = _amat([as_o1, ad_o1, as_s1, ad_s1])
    h1, al1 = _t1_call(xp, w1, am1)

    als1 = jnp.concatenate([al1[:, 0], al1[:, 2]])
    ald1 = jnp.concatenate([al1[:, 1], al1[:, 3]])

    u1, den1 = _edge_kernel(h1.reshape(2 * NP, D), als1, ald1, src, dst)

    b1 = jnp.stack([b_o1, b_s1]).reshape(2, 1, D)
    w2 = jnp.concatenate([W_o2, W_s2], axis=1)
    am2 = _amat([as_o2, ad_o2, as_s2, ad_s2])
    h2, al2 = _t2_call(u1.reshape(2, NP, D), den1.reshape(2, 16, NP),
                       b1, w2, am2)

    als2 = jnp.concatenate([al2[:, 0], al2[:, 2]])
    ald2 = jnp.concatenate([al2[:, 1], al2[:, 3]])

    u2, den2 = _edge_kernel(h2.reshape(2 * NP, D), als2, ald2, src, dst)

    b2 = jnp.stack([b_o2, b_s2]).reshape(2, 1, D)
    out = _t3_call(u2.reshape(2, NP, D), den2.reshape(2, 16, NP),
                   b2, W_pred, b_pred.reshape(1, D))
    return out[:N]


# speedup vs baseline: 1.0353x; 1.0011x over previous
import jax
import jax.numpy as jnp
from jax import lax
from jax.experimental import pallas as pl
from jax.experimental.pallas import tpu as pltpu
from jax.experimental.pallas import tpu_sc as plsc

N = 10000
D = 128
NP = 10240
ES = 320000 + NP
CH = 64
EPT = -(-ES // (16 * 128)) * 128
E_PAD = 16 * EPT
NCH = EPT // CH
SB = 12
B = 1024
GRID = NP // B

_mesh = plsc.VectorSubcoreMesh(core_axis_name="c", subcore_axis_name="s")


def _edge_body(h_hbm, als_hbm, ald_hbm, src_hbm, dst_hbm, u_hbm, den_hbm,
               su, als_v, ald_v, den_v, src_c, dst_c, gidx_v, didx_v0,
               didx_v1, w_v, rows_v0, rows_v1, sem, sem2):
    c = lax.axis_index("c")
    s = lax.axis_index("s")
    wid = c * 16 + s
    coff = c * NP

    pltpu.sync_copy(als_hbm.at[pl.ds(coff, NP)], als_v)
    pltpu.sync_copy(ald_hbm.at[pl.ds(coff, NP)], ald_v)

    zero = jnp.zeros((16,), jnp.float32)

    @pl.loop(0, CH)
    def _(i):
        for j in range(8):
            rows_v0[i, pl.ds(j * 16, 16)] = zero

    @pl.loop(0, NP // 16)
    def _(i):
        den_v[pl.ds(i * 16, 16)] = zero

    @pl.loop(0, NP // 16 // CH)
    def _(i):
        r0 = s * (NP // 16) + i * CH
        pltpu.sync_copy(rows_v0, su.at[pl.ds(r0, CH)])

    plsc.subcore_barrier()

    rows_bufs = (rows_v0, rows_v1)
    didx_bufs = (didx_v0, didx_v1)

    @pl.loop(0, NCH // SB)
    def _(sb):
        sbase = sb * SB * CH
        pltpu.sync_copy(src_hbm.at[wid, pl.ds(sbase, SB * CH)], src_c)
        pltpu.sync_copy(dst_hbm.at[wid, pl.ds(sbase, SB * CH)], dst_c)

        @pl.loop(0, SB // 2)
        def _(p):
            for b in range(2):
                ci = p * 2 + b
                base = ci * CH
                gci = sb * SB + ci
                rows_b = rows_bufs[b]
                didx_b = didx_bufs[b]

                for g in range(CH // 16):
                    off = base + g * 16
                    gidx_v[pl.ds(g * 16, 16)] = src_c[pl.ds(off, 16)] + coff

                @pl.when(gci >= 2)
                def _():
                    pltpu.make_async_copy(rows_b, su.at[didx_b], sem2).wait()

                cp = pltpu.async_copy(h_hbm.at[gidx_v], rows_b, sem)

                for g in range(CH // 16):
                    off = base + g * 16
                    si = src_c[pl.ds(off, 16)]
                    di = dst_c[pl.ds(off, 16)]
                    didx_b[pl.ds(g * 16, 16)] = di
                    a = (plsc.load_gather(als_v, [si])
                         + plsc.load_gather(ald_v, [di]))
                    a = jnp.where(a > 0.0, a, 0.2 * a)
                    w = jnp.exp(a)
                    w_v[pl.ds(g * 16, 16)] = w
                    plsc.addupdate_scatter(den_v, [di], w)

                cp.wait()

                @pl.loop(0, CH, unroll=8)
                def _(e):
                    wv = plsc.load_gather(w_v, [jnp.full((16,), e, jnp.int32)])
                    for j in range(8):
                        rows_b[e, pl.ds(j * 16, 16)] = (
                            rows_b[e, pl.ds(j * 16, 16)] * wv)

                pltpu.async_copy(rows_b, su.at[didx_b], sem2, add=True)

    pltpu.make_async_copy(rows_v0, su.at[didx_v0], sem2).wait()
    pltpu.make_async_copy(rows_v1, su.at[didx_v1], sem2).wait()

    plsc.subcore_barrier()

    @pl.loop(0, NP // 16 // CH)
    def _(i):
        r0 = s * (NP // 16) + i * CH
        pltpu.sync_copy(su.at[pl.ds(r0, CH)], rows_v0)
        pltpu.sync_copy(rows_v0, u_hbm.at[pl.ds(coff + r0, CH)])
    pltpu.sync_copy(den_v, den_hbm.at[wid])


_edge_kernel = pl.kernel(
    _edge_body,
    out_type=[
        jax.ShapeDtypeStruct((2 * NP, D), jnp.float32),
        jax.ShapeDtypeStruct((32, NP), jnp.float32),
    ],
    mesh=_mesh,
    scratch_types=[
        pltpu.VMEM_SHARED((NP, D), jnp.float32),
        pltpu.VMEM((NP,), jnp.float32),
        pltpu.VMEM((NP,), jnp.float32),
        pltpu.VMEM((NP,), jnp.float32),
        pltpu.VMEM((SB * CH,), jnp.int32),
        pltpu.VMEM((SB * CH,), jnp.int32),
        pltpu.VMEM((CH,), jnp.int32),
        pltpu.VMEM((CH,), jnp.int32),
        pltpu.VMEM((CH,), jnp.int32),
        pltpu.VMEM((CH,), jnp.float32),
        pltpu.VMEM((CH, D), jnp.float32),
        pltpu.VMEM((CH, D), jnp.float32),
        pltpu.SemaphoreType.DMA,
        pltpu.SemaphoreType.DMA,
    ],
    compiler_params=pltpu.CompilerParams(needs_layout_passes=False),
)


def _t1_body(x_ref, w_ref, am_ref, h_ref, al_ref):
    h = jnp.dot(x_ref[...], w_ref[...], preferred_element_type=jnp.float32)
    h_ref[0] = h[:, :D]
    h_ref[1] = h[:, D:]
    al_ref[...] = jnp.dot(h, am_ref[...], preferred_element_type=jnp.float32)


def _dcol(den_ref, k):
    return jnp.sum(den_ref[k], axis=0)[:, None] + 1e-16


def _t2_body(u_ref, den_ref, b_ref, w2_ref, am_ref, h_ref, al_ref):
    x1o = jax.nn.relu(u_ref[0] / _dcol(den_ref, 0) + b_ref[0])
    x1s = jax.nn.relu(u_ref[1] / _dcol(den_ref, 1) + b_ref[1])
    x1 = jnp.concatenate([x1o, x1s], axis=1)
    h = jnp.dot(x1, w2_ref[...], preferred_element_type=jnp.float32)
    h_ref[0] = h[:, :D]
    h_ref[1] = h[:, D:]
    al_ref[...] = jnp.dot(h, am_ref[...], preferred_element_type=jnp.float32)


def _t3_body(u_ref, den_ref, b_ref, wp_ref, bp_ref, out_ref):
    x2o = u_ref[0] / _dcol(den_ref, 0) + b_ref[0]
    x2s = u_ref[1] / _dcol(den_ref, 1) + b_ref[1]
    x2 = jnp.concatenate([x2o, x2s], axis=1)
    out_ref[...] = (jnp.dot(x2, wp_ref[...], preferred_element_type=jnp.float32)
                    + bp_ref[...])


_u_spec = pl.BlockSpec((2, B, D), lambda i: (0, i, 0))
_den_spec = pl.BlockSpec((2, 16, B), lambda i: (0, 0, i))
_b_spec = pl.BlockSpec((2, 1, D), lambda i: (0, 0, 0))

_t1_call = pl.pallas_call(
    _t1_body,
    grid=(GRID,),
    in_specs=[
        pl.BlockSpec((B, D), lambda i: (i, 0)),
        pl.BlockSpec((D, 2 * D), lambda i: (0, 0)),
        pl.BlockSpec((2 * D, 8), lambda i: (0, 0)),
    ],
    out_specs=[
        pl.BlockSpec((2, B, D), lambda i: (0, i, 0)),
        pl.BlockSpec((B, 8), lambda i: (i, 0)),
    ],
    out_shape=[
        jax.ShapeDtypeStruct((2, NP, D), jnp.float32),
        jax.ShapeDtypeStruct((NP, 8), jnp.float32),
    ],
)

_t2_call = pl.pallas_call(
    _t2_body,
    grid=(GRID,),
    in_specs=[
        _u_spec, _den_spec, _b_spec,
        pl.BlockSpec((2 * D, 2 * D), lambda i: (0, 0)),
        pl.BlockSpec((2 * D, 8), lambda i: (0, 0)),
    ],
    out_specs=[
        pl.BlockSpec((2, B, D), lambda i: (0, i, 0)),
        pl.BlockSpec((B, 8), lambda i: (i, 0)),
    ],
    out_shape=[
        jax.ShapeDtypeStruct((2, NP, D), jnp.float32),
        jax.ShapeDtypeStruct((NP, 8), jnp.float32),
    ],
)

_t3_call = pl.pallas_call(
    _t3_body,
    grid=(GRID,),
    in_specs=[
        _u_spec, _den_spec, _b_spec,
        pl.BlockSpec((2 * D, D), lambda i: (0, 0)),
        pl.BlockSpec((1, D), lambda i: (0, 0)),
    ],
    out_specs=pl.BlockSpec((B, D), lambda i: (i, 0)),
    out_shape=jax.ShapeDtypeStruct((NP, D), jnp.float32),
)


def _amat(a_list):
    z = jnp.zeros((D,), jnp.float32)
    cols = [jnp.concatenate([a_list[0], z]), jnp.concatenate([a_list[1], z]),
            jnp.concatenate([z, a_list[2]]), jnp.concatenate([z, a_list[3]])]
    cols += [jnp.concatenate([z, z])] * 4
    return jnp.stack(cols, axis=1)


def kernel(x, edge_index_o, edge_index_s, W_o1, as_o1, ad_o1, b_o1,
           W_s1, as_s1, ad_s1, b_s1, W_o2, as_o2, ad_o2, b_o2,
           W_s2, as_s2, ad_s2, b_s2, W_pred, b_pred):
    loops = jnp.arange(NP, dtype=jnp.int32)

    def prep(ei):
        s = jnp.concatenate([ei[0], loops])
        d = jnp.concatenate([ei[1], loops])
        s = jnp.pad(s, (0, E_PAD - ES), constant_values=N)
        d = jnp.pad(d, (0, E_PAD - ES), constant_values=N)
        return s.reshape(16, EPT), d.reshape(16, EPT)

    so, do_ = prep(edge_index_o)
    ss, ds_ = prep(edge_index_s)
    src = jnp.concatenate([so, ss], axis=0)
    dst = jnp.concatenate([do_, ds_], axis=0)

    xp = jnp.pad(x, ((0, NP - N), (0, 0)))

    w1 = jnp.concatenate([W_o1, W_s1], axis=1)
    am1 = _amat([as_o1, ad_o1, as_s1, ad_s1])
    h1, al1 = _t1_call(xp, w1, am1)

    als1 = jnp.concatenate([al1[:, 0], al1[:, 2]])
    ald1 = jnp.concatenate([al1[:, 1], al1[:, 3]])

    u1, den1 = _edge_kernel(h1.reshape(2 * NP, D), als1, ald1, src, dst)

    b1 = jnp.stack([b_o1, b_s1]).reshape(2, 1, D)
    w2 = jnp.concatenate([W_o2, W_s2], axis=1)
    am2 = _amat([as_o2, ad_o2, as_s2, ad_s2])
    h2, al2 = _t2_call(u1.reshape(2, NP, D), den1.reshape(2, 16, NP),
                       b1, w2, am2)

    als2 = jnp.concatenate([al2[:, 0], al2[:, 2]])
    ald2 = jnp.concatenate([al2[:, 1], al2[:, 3]])

    u2, den2 = _edge_kernel(h2.reshape(2 * NP, D), als2, ald2, src, dst)

    b2 = jnp.stack([b_o2, b_s2]).reshape(2, 1, D)
    out = _t3_call(u2.reshape(2, NP, D), den2.reshape(2, 16, NP),
                   b2, W_pred, b_pred.reshape(1, D))
    return out[:N]
